# baseline (speedup 1.0000x reference)
import functools

import jax
import jax.numpy as jnp
from jax import lax
from jax.experimental import pallas as pl
from jax.experimental.pallas import tpu as pltpu
from jax.experimental.pallas import tpu_sc as plsc

N = 10000
E = 160000
M = 9
C = 128
H = 128
D = M * C

BN = 1000
BE = 640
PLANE_L = (0, 1, 1, 1, 2, 2, 2, 2, 2)
L_OFF = ((0, 1), (1, 3), (4, 5))

NC = 2
NS = 16
NW = NC * NS
EW = E // NW
GW = 40
NWIN = EW // GW

CB = 64
NRND = 5
OUTROWS = NRND * NW * CB
RW = 2000
CAP = 1536
ACR = CB + 8


def _l_norm_planes(planes, g_ref):
    out = [None] * 9
    for l, (lo, sz) in enumerate(L_OFF):
        ss = None
        for p in range(lo, lo + sz):
            s = jnp.sum(planes[p] * planes[p], axis=1, keepdims=True)
            ss = s if ss is None else ss + s
        inv = lax.rsqrt(ss / (sz * C) + 1e-6)
        gl = g_ref[l:l + 1, :]
        for p in range(lo, lo + sz):
            out[p] = planes[p] * inv * gl
    return out


def _pre_body(nf_ref, g1_ref, ws_ref, wd_ref, a_ref, b_ref):
    ws = ws_ref[...]
    wd = wd_ref[...]
    planes = [nf_ref[:, i * C:(i + 1) * C] for i in range(9)]
    xn = _l_norm_planes(planes, g1_ref)
    for p in range(9):
        a_ref[:, p * C:(p + 1) * C] = jnp.dot(
            xn[p], ws, preferred_element_type=jnp.float32)
        b_ref[:, p * C:(p + 1) * C] = jnp.dot(
            xn[p], wd, preferred_element_type=jnp.float32)


def _edge_body(a_ref, b_ref, wig_ref, ee_ref, env_ref,
               we1_ref, we2_ref, out_ref):
    u = jnp.dot(ee_ref[...], we1_ref[...], preferred_element_type=jnp.float32)
    u = u * jax.nn.sigmoid(u)
    g = jax.nn.sigmoid(jnp.dot(u, we2_ref[...],
                               preferred_element_type=jnp.float32))
    gt = jnp.transpose(g)
    wig = wig_ref[...]
    t = [jnp.transpose(a_ref[:, i * C:(i + 1) * C]
                       + b_ref[:, i * C:(i + 1) * C]) for i in range(9)]
    h = []
    for m in range(9):
        acc = None
        for n in range(9):
            w = wig[m * 9 + n:m * 9 + n + 1, :]
            acc = w * t[n] if acc is None else acc + w * t[n]
        h.append(acc * gt)
    sig0 = jax.nn.sigmoid(h[0])
    hh = [h[0] * sig0] + [h[m] * sig0 for m in range(1, 9)]
    env = env_ref[...]
    for m in range(9):
        acc = None
        for n in range(9):
            w = wig[n * 9 + m:n * 9 + m + 1, :]
            acc = w * hh[n] if acc is None else acc + w * hh[n]
        out_ref[:, m * C:(m + 1) * C] = jnp.transpose(acc * env)


def _post_body(nf_ref, agg_ref, wmsg_ref, g2_ref, wf1_ref, wf2_ref, out_ref):
    wmsg = wmsg_ref[...]
    x2 = [nf_ref[:, p * C:(p + 1) * C] +
          jnp.dot(agg_ref[:, p * C:(p + 1) * C], wmsg,
                  preferred_element_type=jnp.float32)
          for p in range(9)]
    y = _l_norm_planes(x2, g2_ref)
    hf = [jnp.dot(y[p], wf1_ref[PLANE_L[p] * C:(PLANE_L[p] + 1) * C, :],
                  preferred_element_type=jnp.float32) for p in range(9)]
    sig0 = jax.nn.sigmoid(hf[0])
    hh = [hf[0] * sig0] + [hf[p] * sig0 for p in range(1, 9)]
    for p in range(9):
        out_ref[:, p * C:(p + 1) * C] = x2[p] + jnp.dot(
            hh[p], wf2_ref[PLANE_L[p] * C:(PLANE_L[p] + 1) * C, :],
            preferred_element_type=jnp.float32)


def _gather_sc(a_hbm, b_hbm, s_hbm, r_hbm, ao_hbm, bo_hbm,
               sidx, ab0, ab1, sa0, sa1):
    wid = lax.axis_index("s") * NC + lax.axis_index("c")
    base = wid * EW

    def issue(table, w, buf, sem):
        pltpu.async_copy(table.at[sidx.at[pl.ds(w * GW, GW)]], buf, sem)

    def drain(buf, sem):
        pltpu.make_async_copy(a_hbm.at[pl.ds(0, GW)], buf, sem).wait()

    def put(out, w, buf):
        pltpu.sync_copy(buf, out.at[pl.ds(base + w * GW, GW)])

    for table, idx_hbm, out in ((a_hbm, s_hbm, ao_hbm),
                                (b_hbm, r_hbm, bo_hbm)):
        pltpu.sync_copy(idx_hbm.at[pl.ds(base, EW)], sidx)
        issue(table, 0, ab0, sa0)

        def pair(p, carry, _t=table, _o=out):
            w0 = 2 * p
            issue(_t, w0 + 1, ab1, sa1)
            drain(ab0, sa0)
            put(_o, w0, ab0)
            issue(_t, w0 + 2, ab0, sa0)
            drain(ab1, sa1)
            put(_o, w0 + 1, ab1)
            return carry

        lax.fori_loop(0, NWIN // 2, pair, 0)
        drain(ab0, sa0)
        put(out, NWIN - 1, ab0)


def _scatter_sc(r_hbm, recv_hbm, zeros_hbm, agg_hbm,
                rcvwin, rb0, rb1, pklist,
                cnt_ref, accum, gs0, gs1):
    wid = lax.axis_index("s") * NC + lax.axis_index("c")

    lanes = lax.iota(jnp.int32, 16)

    for k in range(NRND):
        cnt_ref[k] = 0

    def outer_body(w, carry0):
        pltpu.sync_copy(recv_hbm.at[pl.ds(w * RW, RW)], rcvwin)
        ebase = w * RW

        def scan_body(i, carry):
            rv = rcvwin[pl.ds(i * 16, 16)]
            bv = lax.shift_right_logical(rv, 6)
            hit = lax.bitwise_and(bv, NW - 1) == wid
            nhit = jnp.sum(hit.astype(jnp.int32))

            @pl.when(nhit > 0)
            def _():
                ev = ebase + i * 16 + lanes
                packed = ev * 128 + lax.bitwise_and(rv, CB - 1)
                for k in range(NRND):
                    mask = bv == (k * NW + wid)
                    mi = mask.astype(jnp.int32)
                    ck = cnt_ref[k]
                    pos = k * CAP + ck + plsc.cumsum(mi) - 1
                    plsc.store_scatter(pklist, [pos], packed, mask=mask)
                    cnt_ref[k] = ck + jnp.sum(mi)

            return carry

        return lax.fori_loop(0, RW // 16, scan_body, carry0)

    lax.fori_loop(0, E // RW, outer_body, 0)

    ones = lanes < 16
    pad_loc = CB + lax.bitwise_and(lanes, 7)
    for k in range(NRND):
        ck = cnt_ref[k]
        for q in range(4):
            plsc.store_scatter(pklist, [k * CAP + ck + q * 16 + lanes],
                               pad_loc, mask=ones)

    def load_pk(kbase, w):
        return pklist[pl.ds(kbase + w * 16, 16)]

    def issue(kbase, w, buf, sem):
        ev = lax.shift_right_logical(load_pk(kbase, w), 7)
        pltpu.async_copy(r_hbm.at[ev], buf, sem)

    def drain(buf, sem):
        pltpu.make_async_copy(r_hbm.at[pl.ds(0, 16)], buf, sem).wait()

    def consume(kbase, w, buf):
        pkv = load_pk(kbase, w)
        for j in range(16):
            lv = lax.bitwise_and(pkv[j], 127)
            for j2 in range(D // 16):
                plsc.addupdate(accum.at[lv, pl.ds(j2 * 16, 16)],
                               buf[j, pl.ds(j2 * 16, 16)])

    def round_body(k, carry0):
        chunk_base = (k * NW + wid) * CB
        kbase = k * CAP
        pltpu.sync_copy(zeros_hbm, accum)

        npair = cnt_ref[k] // 32 + 1
        issue(kbase, 0, rb0, gs0)

        def pair_body(p, carry):
            issue(kbase, 2 * p + 1, rb1, gs1)
            drain(rb0, gs0)
            consume(kbase, 2 * p, rb0)
            issue(kbase, 2 * p + 2, rb0, gs0)
            drain(rb1, gs1)
            consume(kbase, 2 * p + 1, rb1)
            return carry

        lax.fori_loop(0, npair, pair_body, 0)
        drain(rb0, gs0)
        pltpu.sync_copy(accum.at[pl.ds(0, CB)],
                        agg_hbm.at[pl.ds(chunk_base, CB)])
        return carry0

    lax.fori_loop(0, NRND, round_body, 0)


def _pre_call(nf2, gamma1, W_src, W_dst):
    return pl.pallas_call(
        _pre_body,
        grid=(N // BN,),
        in_specs=[
            pl.BlockSpec((BN, D), lambda i: (i, 0)),
            pl.BlockSpec((3, C), lambda i: (0, 0)),
            pl.BlockSpec((C, H), lambda i: (0, 0)),
            pl.BlockSpec((C, H), lambda i: (0, 0)),
        ],
        out_specs=[pl.BlockSpec((BN, D), lambda i: (i, 0))] * 2,
        out_shape=[jax.ShapeDtypeStruct((N, D), jnp.float32)] * 2,
    )(nf2, gamma1, W_src, W_dst)


@functools.lru_cache(maxsize=1)
def _sc_calls():
    mesh = plsc.VectorSubcoreMesh(core_axis_name="c", subcore_axis_name="s")
    gather_call = pl.kernel(
        _gather_sc,
        out_type=[jax.ShapeDtypeStruct((E, D), jnp.float32)] * 2,
        mesh=mesh,
        scratch_types=[
            pltpu.VMEM((EW,), jnp.int32),
            pltpu.VMEM((GW, D), jnp.float32),
            pltpu.VMEM((GW, D), jnp.float32),
            pltpu.SemaphoreType.DMA,
            pltpu.SemaphoreType.DMA,
        ],
    )
    scatter_call = pl.kernel(
        _scatter_sc,
        out_type=jax.ShapeDtypeStruct((OUTROWS, D), jnp.float32),
        mesh=mesh,
        scratch_types=(
            [
                pltpu.VMEM((RW,), jnp.int32),
                pltpu.VMEM((16, D), jnp.float32),
                pltpu.VMEM((16, D), jnp.float32),
                pltpu.VMEM((NRND * CAP,), jnp.int32),
            ]
            + [
                pltpu.SMEM((8,), jnp.int32),
                pltpu.VMEM((ACR, D), jnp.float32),
                pltpu.SemaphoreType.DMA,
                pltpu.SemaphoreType.DMA,
            ]
        ),
        compiler_params=pltpu.CompilerParams(needs_layout_passes=False),
    )
    return gather_call, scatter_call


def _edge_call(A_e, B_e, wigc, edge_embeds, env_row, W_e1, W_e2):
    return pl.pallas_call(
        _edge_body,
        grid=(E // BE,),
        in_specs=[
            pl.BlockSpec((BE, D), lambda i: (i, 0)),
            pl.BlockSpec((BE, D), lambda i: (i, 0)),
            pl.BlockSpec((81, BE), lambda i: (0, i)),
            pl.BlockSpec((BE, C), lambda i: (i, 0)),
            pl.BlockSpec((1, BE), lambda i: (0, i)),
            pl.BlockSpec((C, H), lambda i: (0, 0)),
            pl.BlockSpec((H, H), lambda i: (0, 0)),
        ],
        out_specs=pl.BlockSpec((BE, D), lambda i: (i, 0)),
        out_shape=jax.ShapeDtypeStruct((E, D), jnp.float32),
    )(A_e, B_e, wigc, edge_embeds, env_row, W_e1, W_e2)


def _post_call(nf2, agg, W_msg, gamma2, wf1, wf2):
    return pl.pallas_call(
        _post_body,
        grid=(N // BN,),
        in_specs=[
            pl.BlockSpec((BN, D), lambda i: (i, 0)),
            pl.BlockSpec((BN, D), lambda i: (i, 0)),
            pl.BlockSpec((H, C), lambda i: (0, 0)),
            pl.BlockSpec((3, C), lambda i: (0, 0)),
            pl.BlockSpec((3 * C, H), lambda i: (0, 0)),
            pl.BlockSpec((3 * H, C), lambda i: (0, 0)),
        ],
        out_specs=pl.BlockSpec((BN, D), lambda i: (i, 0)),
        out_shape=jax.ShapeDtypeStruct((N, D), jnp.float32),
    )(nf2, agg, W_msg, gamma2, wf1, wf2)


def kernel(node_feats, edge_embeds, wigner, edge_envelope, gamma1, gamma2,
           W_src, W_dst, W_e1, W_e2, W_msg, W_ff1, W_ff2, senders, receivers):
    nf2 = node_feats.reshape(N, D)
    wigc = jnp.transpose(wigner.reshape(E, 81))
    env_row = edge_envelope.reshape(1, E)
    snd = senders.astype(jnp.int32)
    rcv = receivers.astype(jnp.int32)
    zacc = jnp.zeros((ACR, D), jnp.float32)

    gather_call, scatter_call = _sc_calls()
    a, b = _pre_call(nf2, gamma1, W_src, W_dst)
    A_e, B_e = gather_call(a, b, snd, rcv)
    r = _edge_call(A_e, B_e, wigc, edge_embeds, env_row, W_e1, W_e2)
    agg_full = scatter_call(r, rcv, zacc)
    agg = agg_full[:N]
    out2 = _post_call(nf2, agg, W_msg, gamma2,
                      W_ff1.reshape(3 * C, H), W_ff2.reshape(3 * H, C))
    return out2.reshape(N, M, C)


# speedup vs baseline: 1.0633x; 1.0633x over previous
import functools

import jax
import jax.numpy as jnp
from jax import lax
from jax.experimental import pallas as pl
from jax.experimental.pallas import tpu as pltpu
from jax.experimental.pallas import tpu_sc as plsc

N = 10000
E = 160000
M = 9
C = 128
H = 128
D = M * C

BN = 1000
BE = 640
PLANE_L = (0, 1, 1, 1, 2, 2, 2, 2, 2)
L_OFF = ((0, 1), (1, 3), (4, 5))

NC = 2
NS = 16
NW = NC * NS
EW = E // NW
GW = 40
NWIN = EW // GW

CB = 64
NRND = 5
OUTROWS = NRND * NW * CB
RW = 2000
CAP = 1536
ACR = CB + 8


def _l_norm_planes(planes, g_ref):
    out = [None] * 9
    for l, (lo, sz) in enumerate(L_OFF):
        ss = None
        for p in range(lo, lo + sz):
            s = jnp.sum(planes[p] * planes[p], axis=1, keepdims=True)
            ss = s if ss is None else ss + s
        inv = lax.rsqrt(ss / (sz * C) + 1e-6)
        gl = g_ref[l:l + 1, :]
        for p in range(lo, lo + sz):
            out[p] = planes[p] * inv * gl
    return out


def _pre_body(nf_ref, g1_ref, ws_ref, wd_ref, a_ref, b_ref):
    ws = ws_ref[...]
    wd = wd_ref[...]
    planes = [nf_ref[:, i * C:(i + 1) * C] for i in range(9)]
    xn = _l_norm_planes(planes, g1_ref)
    for p in range(9):
        a_ref[:, p * C:(p + 1) * C] = jnp.dot(
            xn[p], ws, preferred_element_type=jnp.float32)
        b_ref[:, p * C:(p + 1) * C] = jnp.dot(
            xn[p], wd, preferred_element_type=jnp.float32)


def _edge_body(a_ref, b_ref, wig_ref, ee_ref, env_ref,
               we1_ref, we2_ref, out_ref):
    u = jnp.dot(ee_ref[...], we1_ref[...], preferred_element_type=jnp.float32)
    u = u * jax.nn.sigmoid(u)
    g = jax.nn.sigmoid(jnp.dot(u, we2_ref[...],
                               preferred_element_type=jnp.float32))
    gt = jnp.transpose(g)
    wig = wig_ref[...]
    t = [jnp.transpose(a_ref[:, i * C:(i + 1) * C]
                       + b_ref[:, i * C:(i + 1) * C]) for i in range(9)]
    h = []
    for m in range(9):
        acc = None
        for n in range(9):
            w = wig[m * 9 + n:m * 9 + n + 1, :]
            acc = w * t[n] if acc is None else acc + w * t[n]
        h.append(acc * gt)
    sig0 = jax.nn.sigmoid(h[0])
    hh = [h[0] * sig0] + [h[m] * sig0 for m in range(1, 9)]
    env = env_ref[...]
    for m in range(9):
        acc = None
        for n in range(9):
            w = wig[n * 9 + m:n * 9 + m + 1, :]
            acc = w * hh[n] if acc is None else acc + w * hh[n]
        out_ref[:, m * C:(m + 1) * C] = jnp.transpose(acc * env)


def _post_body(nf_ref, agg_ref, wmsg_ref, g2_ref, wf1_ref, wf2_ref, out_ref):
    wmsg = wmsg_ref[...]
    x2 = [nf_ref[:, p * C:(p + 1) * C] +
          jnp.dot(agg_ref[:, p * C:(p + 1) * C], wmsg,
                  preferred_element_type=jnp.float32)
          for p in range(9)]
    y = _l_norm_planes(x2, g2_ref)
    hf = [jnp.dot(y[p], wf1_ref[PLANE_L[p] * C:(PLANE_L[p] + 1) * C, :],
                  preferred_element_type=jnp.float32) for p in range(9)]
    sig0 = jax.nn.sigmoid(hf[0])
    hh = [hf[0] * sig0] + [hf[p] * sig0 for p in range(1, 9)]
    for p in range(9):
        out_ref[:, p * C:(p + 1) * C] = x2[p] + jnp.dot(
            hh[p], wf2_ref[PLANE_L[p] * C:(PLANE_L[p] + 1) * C, :],
            preferred_element_type=jnp.float32)


def _gather_sc(a_hbm, b_hbm, s_hbm, r_hbm, ao_hbm, bo_hbm,
               sidx, ab0, ab1, sa0, sa1):
    wid = lax.axis_index("s") * NC + lax.axis_index("c")
    base = wid * EW

    def issue(table, w, buf, sem):
        pltpu.async_copy(table.at[sidx.at[pl.ds(w * GW, GW)]], buf, sem)

    def drain(buf, sem):
        pltpu.make_async_copy(a_hbm.at[pl.ds(0, GW)], buf, sem).wait()

    def put(out, w, buf):
        pltpu.sync_copy(buf, out.at[pl.ds(base + w * GW, GW)])

    for table, idx_hbm, out in ((a_hbm, s_hbm, ao_hbm),
                                (b_hbm, r_hbm, bo_hbm)):
        pltpu.sync_copy(idx_hbm.at[pl.ds(base, EW)], sidx)
        issue(table, 0, ab0, sa0)

        def pair(p, carry, _t=table, _o=out):
            w0 = 2 * p
            issue(_t, w0 + 1, ab1, sa1)
            drain(ab0, sa0)
            put(_o, w0, ab0)
            issue(_t, w0 + 2, ab0, sa0)
            drain(ab1, sa1)
            put(_o, w0 + 1, ab1)
            return carry

        lax.fori_loop(0, NWIN // 2, pair, 0)
        drain(ab0, sa0)
        put(out, NWIN - 1, ab0)


def _scatter_sc(r_hbm, recv_hbm, zeros_hbm, agg_hbm,
                rcvwin, rb0, rb1, pk0, pk1, pk2, pk3, pk4,
                cnt_ref, accum, gs0, gs1):
    wid = lax.axis_index("s") * NC + lax.axis_index("c")
    pks = (pk0, pk1, pk2, pk3, pk4)

    lanes = lax.iota(jnp.int32, 16)

    for k in range(NRND):
        cnt_ref[k] = 0

    def outer_body(w, carry0):
        pltpu.sync_copy(recv_hbm.at[pl.ds(w * RW, RW)], rcvwin)
        ebase = w * RW

        def scan_body(i, carry):
            rv = rcvwin[pl.ds(i * 16, 16)]
            bv = lax.shift_right_logical(rv, 6)
            hit = lax.bitwise_and(bv, NW - 1) == wid
            nhit = jnp.sum(hit.astype(jnp.int32))

            @pl.when(nhit > 0)
            def _():
                ev = ebase + i * 16 + lanes
                packed = ev * 128 + lax.bitwise_and(rv, CB - 1)
                for k in range(NRND):
                    mask = bv == (k * NW + wid)
                    mi = mask.astype(jnp.int32)
                    ck = cnt_ref[k]
                    pos = ck + plsc.cumsum(mi) - 1
                    plsc.store_scatter(pks[k], [pos], packed, mask=mask)
                    cnt_ref[k] = ck + jnp.sum(mi)

            return carry

        return lax.fori_loop(0, RW // 16, scan_body, carry0)

    lax.fori_loop(0, E // RW, outer_body, 0)

    ones = lanes < 16
    pad_loc = CB + lax.bitwise_and(lanes, 7)
    for k in range(NRND):
        ck = cnt_ref[k]
        for q in range(4):
            plsc.store_scatter(pks[k], [ck + q * 16 + lanes], pad_loc,
                               mask=ones)

    def issue(pk, w, buf, sem):
        pkv = pk[pl.ds(w * 16, 16)]
        ev = lax.shift_right_logical(pkv, 7)
        pltpu.async_copy(r_hbm.at[ev], buf, sem)

    def drain(buf, sem):
        pltpu.make_async_copy(r_hbm.at[pl.ds(0, 16)], buf, sem).wait()

    for k in range(NRND):
        chunk_base = (k * NW + wid) * CB
        pltpu.sync_copy(zeros_hbm, accum)

        def consume(w, buf, _k=k):
            def row_body(j, cc):
                pk1 = pks[_k][pl.ds(w * 16 + j, 16)][0]
                lv = lax.bitwise_and(pk1, 127)
                for j2 in range(D // 16):
                    plsc.addupdate(accum.at[lv, pl.ds(j2 * 16, 16)],
                                   buf[j, pl.ds(j2 * 16, 16)])
                return cc
            lax.fori_loop(0, 16, row_body, 0)

        npair = cnt_ref[k] // 32 + 1
        issue(pks[k], 0, rb0, gs0)

        def pair_body(p, carry, _k=k):
            issue(pks[_k], 2 * p + 1, rb1, gs1)
            drain(rb0, gs0)
            consume(2 * p, rb0, _k)
            issue(pks[_k], 2 * p + 2, rb0, gs0)
            drain(rb1, gs1)
            consume(2 * p + 1, rb1, _k)
            return carry

        lax.fori_loop(0, npair, pair_body, 0)
        drain(rb0, gs0)
        pltpu.sync_copy(accum.at[pl.ds(0, CB)],
                        agg_hbm.at[pl.ds(chunk_base, CB)])


def _pre_call(nf2, gamma1, W_src, W_dst):
    return pl.pallas_call(
        _pre_body,
        grid=(N // BN,),
        in_specs=[
            pl.BlockSpec((BN, D), lambda i: (i, 0)),
            pl.BlockSpec((3, C), lambda i: (0, 0)),
            pl.BlockSpec((C, H), lambda i: (0, 0)),
            pl.BlockSpec((C, H), lambda i: (0, 0)),
        ],
        out_specs=[pl.BlockSpec((BN, D), lambda i: (i, 0))] * 2,
        out_shape=[jax.ShapeDtypeStruct((N, D), jnp.float32)] * 2,
    )(nf2, gamma1, W_src, W_dst)


@functools.lru_cache(maxsize=1)
def _sc_calls():
    mesh = plsc.VectorSubcoreMesh(core_axis_name="c", subcore_axis_name="s")
    gather_call = pl.kernel(
        _gather_sc,
        out_type=[jax.ShapeDtypeStruct((E, D), jnp.float32)] * 2,
        mesh=mesh,
        scratch_types=[
            pltpu.VMEM((EW,), jnp.int32),
            pltpu.VMEM((GW, D), jnp.float32),
            pltpu.VMEM((GW, D), jnp.float32),
            pltpu.SemaphoreType.DMA,
            pltpu.SemaphoreType.DMA,
        ],
    )
    scatter_call = pl.kernel(
        _scatter_sc,
        out_type=jax.ShapeDtypeStruct((OUTROWS, D), jnp.float32),
        mesh=mesh,
        scratch_types=(
            [
                pltpu.VMEM((RW,), jnp.int32),
                pltpu.VMEM((16, D), jnp.float32),
                pltpu.VMEM((16, D), jnp.float32),
            ]
            + [pltpu.VMEM((CAP,), jnp.int32)] * NRND
            + [
                pltpu.SMEM((8,), jnp.int32),
                pltpu.VMEM((ACR, D), jnp.float32),
                pltpu.SemaphoreType.DMA,
                pltpu.SemaphoreType.DMA,
            ]
        ),
        compiler_params=pltpu.CompilerParams(needs_layout_passes=False),
    )
    return gather_call, scatter_call


def _edge_call(A_e, B_e, wigc, edge_embeds, env_row, W_e1, W_e2):
    return pl.pallas_call(
        _edge_body,
        grid=(E // BE,),
        in_specs=[
            pl.BlockSpec((BE, D), lambda i: (i, 0)),
            pl.BlockSpec((BE, D), lambda i: (i, 0)),
            pl.BlockSpec((81, BE), lambda i: (0, i)),
            pl.BlockSpec((BE, C), lambda i: (i, 0)),
            pl.BlockSpec((1, BE), lambda i: (0, i)),
            pl.BlockSpec((C, H), lambda i: (0, 0)),
            pl.BlockSpec((H, H), lambda i: (0, 0)),
        ],
        out_specs=pl.BlockSpec((BE, D), lambda i: (i, 0)),
        out_shape=jax.ShapeDtypeStruct((E, D), jnp.float32),
    )(A_e, B_e, wigc, edge_embeds, env_row, W_e1, W_e2)


def _post_call(nf2, agg, W_msg, gamma2, wf1, wf2):
    return pl.pallas_call(
        _post_body,
        grid=(N // BN,),
        in_specs=[
            pl.BlockSpec((BN, D), lambda i: (i, 0)),
            pl.BlockSpec((BN, D), lambda i: (i, 0)),
            pl.BlockSpec((H, C), lambda i: (0, 0)),
            pl.BlockSpec((3, C), lambda i: (0, 0)),
            pl.BlockSpec((3 * C, H), lambda i: (0, 0)),
            pl.BlockSpec((3 * H, C), lambda i: (0, 0)),
        ],
        out_specs=pl.BlockSpec((BN, D), lambda i: (i, 0)),
        out_shape=jax.ShapeDtypeStruct((N, D), jnp.float32),
    )(nf2, agg, W_msg, gamma2, wf1, wf2)


def kernel(node_feats, edge_embeds, wigner, edge_envelope, gamma1, gamma2,
           W_src, W_dst, W_e1, W_e2, W_msg, W_ff1, W_ff2, senders, receivers):
    nf2 = node_feats.reshape(N, D)
    wigc = jnp.transpose(wigner.reshape(E, 81))
    env_row = edge_envelope.reshape(1, E)
    snd = senders.astype(jnp.int32)
    rcv = receivers.astype(jnp.int32)
    zacc = jnp.zeros((ACR, D), jnp.float32)

    gather_call, scatter_call = _sc_calls()
    a, b = _pre_call(nf2, gamma1, W_src, W_dst)
    A_e, B_e = gather_call(a, b, snd, rcv)
    r = _edge_call(A_e, B_e, wigc, edge_embeds, env_row, W_e1, W_e2)
    agg_full = scatter_call(r, rcv, zacc)
    agg = agg_full[:N]
    out2 = _post_call(nf2, agg, W_msg, gamma2,
                      W_ff1.reshape(3 * C, H), W_ff2.reshape(3 * H, C))
    return out2.reshape(N, M, C)
